# baseline (speedup 1.0000x reference)
import functools

import jax
import jax.numpy as jnp
from jax import lax
from jax.experimental import pallas as pl
from jax.experimental.pallas import tpu as pltpu
from jax.experimental.pallas import tpu_sc as plsc

B, S, D = 2, 2048, 2048
E, TOPK, DFF = 8, 2, 1408
T = B * S
N_ENTRIES = T * TOPK
BLK_M = 256
P_MAX = N_ENTRIES + E * BLK_M
NUM_M = P_MAX // BLK_M
NC, NS = 2, 16
NW = NC * NS
GATHER_CH = 40
COMB_CH = 16


def _router_logits(x, gate_w_pad):
    blk = 512

    def body(x_ref, gw_ref, out_ref):
        out_ref[...] = lax.dot_general(
            x_ref[...], gw_ref[...], (((1,), (1,)), ((), ())),
            preferred_element_type=jnp.float32)

    return pl.pallas_call(
        body,
        grid=(T // blk,),
        in_specs=[
            pl.BlockSpec((blk, D), lambda m: (m, 0)),
            pl.BlockSpec((128, D), lambda m: (0, 0)),
        ],
        out_specs=pl.BlockSpec((blk, 128), lambda m: (m, 0)),
        out_shape=jax.ShapeDtypeStruct((T, 128), jnp.float32),
    )(x, gate_w_pad)


def _sc_gather_rows(x, row_ids):
    per_w = P_MAX // NW
    mesh = plsc.VectorSubcoreMesh(core_axis_name="c", subcore_axis_name="s")

    @functools.partial(
        pl.kernel,
        out_type=jax.ShapeDtypeStruct((P_MAX, D), jnp.float32),
        mesh=mesh,
        scratch_types=[
            pltpu.VMEM((GATHER_CH,), jnp.int32),
            pltpu.VMEM((GATHER_CH, D), jnp.float32),
            pltpu.SemaphoreType.DMA,
        ],
    )
    def k(x_hbm, idx_hbm, out_hbm, idx_v, rows_v, sem):
        wid = lax.axis_index("s") * NC + lax.axis_index("c")
        base = wid * per_w

        def body(c, carry):
            off = base + c * GATHER_CH
            pltpu.sync_copy(idx_hbm.at[pl.ds(off, GATHER_CH)], idx_v)
            pltpu.async_copy(x_hbm.at[idx_v], rows_v, sem).wait()
            pltpu.sync_copy(rows_v, out_hbm.at[pl.ds(off, GATHER_CH)])
            return carry

        lax.fori_loop(0, per_w // GATHER_CH, body, 0)

    return k(x, row_ids)


def _sc_combine(out_s, pos0, pos1):
    per_w = T // NW
    mesh = plsc.VectorSubcoreMesh(core_axis_name="c", subcore_axis_name="s")

    @functools.partial(
        pl.kernel,
        out_type=jax.ShapeDtypeStruct((T, D), jnp.float32),
        mesh=mesh,
        scratch_types=[
            pltpu.VMEM((COMB_CH,), jnp.int32),
            pltpu.VMEM((COMB_CH,), jnp.int32),
            pltpu.VMEM((COMB_CH, D), jnp.float32),
            pltpu.VMEM((COMB_CH, D), jnp.float32),
            pltpu.SemaphoreType.DMA,
        ],
    )
    def k(outs_hbm, p0_hbm, p1_hbm, fin_hbm, p0_v, p1_v, a_v, b_v, sem):
        wid = lax.axis_index("s") * NC + lax.axis_index("c")
        base = wid * per_w

        def body(c, carry):
            off = base + c * COMB_CH
            pltpu.sync_copy(p0_hbm.at[pl.ds(off, COMB_CH)], p0_v)
            pltpu.sync_copy(p1_hbm.at[pl.ds(off, COMB_CH)], p1_v)
            pltpu.async_copy(outs_hbm.at[p0_v], a_v, sem).wait()
            pltpu.async_copy(outs_hbm.at[p1_v], b_v, sem).wait()

            def row(r, rc):
                def col(cc, cyc):
                    sl = pl.ds(cc * 16, 16)
                    a_v[r, sl] = a_v[r, sl] + b_v[r, sl]
                    return cyc
                return lax.fori_loop(0, D // 16, col, rc)

            lax.fori_loop(0, COMB_CH, row, 0)
            pltpu.sync_copy(a_v, fin_hbm.at[pl.ds(off, COMB_CH)])
            return carry

        lax.fori_loop(0, per_w // COMB_CH, body, 0)

    return k(out_s, pos0, pos1)


def _grouped_gate_up(x_s, gate_proj_w, up_proj_w, wrow_r, gid, act):
    def body(gid_ref, act_ref, xs_ref, wg_ref, wu_ref, wr_ref, h_ref,
             wgb_ref, wub_ref, last_ref):
        m = pl.program_id(0)

        @pl.when(m == 0)
        def _():
            last_ref[0] = -1

        @pl.when(act_ref[m] == 1)
        def _():
            @pl.when(last_ref[0] != gid_ref[m])
            def _():
                wgb_ref[...] = wg_ref[0].astype(jnp.bfloat16)
                wub_ref[...] = wu_ref[0].astype(jnp.bfloat16)
                last_ref[0] = gid_ref[m]

            x = xs_ref[...].astype(jnp.bfloat16)
            g = lax.dot_general(x, wgb_ref[...], (((1,), (1,)), ((), ())),
                                preferred_element_type=jnp.float32)
            u = lax.dot_general(x, wub_ref[...], (((1,), (1,)), ((), ())),
                                preferred_element_type=jnp.float32)
            h = (g * jax.nn.sigmoid(g)) * u
            h_ref[...] = (h * wr_ref[0, 0, :][:, None]).astype(jnp.bfloat16)

    grid_spec = pltpu.PrefetchScalarGridSpec(
        num_scalar_prefetch=2,
        grid=(NUM_M,),
        in_specs=[
            pl.BlockSpec(
                (BLK_M, D),
                lambda m, gid, act: (jnp.where(act[m] == 1, m, 0), 0)),
            pl.BlockSpec((1, DFF, D), lambda m, gid, act: (gid[m], 0, 0)),
            pl.BlockSpec((1, DFF, D), lambda m, gid, act: (gid[m], 0, 0)),
            pl.BlockSpec((1, 1, BLK_M), lambda m, gid, act: (m, 0, 0)),
        ],
        out_specs=pl.BlockSpec((BLK_M, DFF), lambda m, gid, act: (m, 0)),
        scratch_shapes=[
            pltpu.VMEM((DFF, D), jnp.bfloat16),
            pltpu.VMEM((DFF, D), jnp.bfloat16),
            pltpu.SMEM((1,), jnp.int32),
        ],
    )
    return pl.pallas_call(
        body,
        grid_spec=grid_spec,
        out_shape=jax.ShapeDtypeStruct((P_MAX, DFF), jnp.bfloat16),
        compiler_params=pltpu.CompilerParams(vmem_limit_bytes=100 * 1024 * 1024),
    )(gid, act, x_s, gate_proj_w, up_proj_w, wrow_r)


def _grouped_down(h, down_proj_w, gid, act):
    def body(gid_ref, act_ref, h_ref, wd_ref, out_ref, wdb_ref, last_ref):
        m = pl.program_id(0)

        @pl.when(m == 0)
        def _():
            last_ref[0] = -1

        @pl.when(act_ref[m] == 1)
        def _():
            @pl.when(last_ref[0] != gid_ref[m])
            def _():
                wdb_ref[...] = wd_ref[0].astype(jnp.bfloat16)
                last_ref[0] = gid_ref[m]

            out_ref[...] = lax.dot_general(
                h_ref[...], wdb_ref[...], (((1,), (1,)), ((), ())),
                preferred_element_type=jnp.float32)

    grid_spec = pltpu.PrefetchScalarGridSpec(
        num_scalar_prefetch=2,
        grid=(NUM_M,),
        in_specs=[
            pl.BlockSpec((BLK_M, DFF),
                         lambda m, gid, act: (jnp.where(act[m] == 1, m, 0), 0)),
            pl.BlockSpec((1, D, DFF), lambda m, gid, act: (gid[m], 0, 0)),
        ],
        out_specs=pl.BlockSpec((BLK_M, D), lambda m, gid, act: (m, 0)),
        scratch_shapes=[
            pltpu.VMEM((D, DFF), jnp.bfloat16),
            pltpu.SMEM((1,), jnp.int32),
        ],
    )
    return pl.pallas_call(
        body,
        grid_spec=grid_spec,
        out_shape=jax.ShapeDtypeStruct((P_MAX, D), jnp.float32),
    )(gid, act, h, down_proj_w)


def kernel(hidden_states, gate_w, gate_proj_w, up_proj_w, down_proj_w):
    x = hidden_states.reshape(-1, D)

    gate_w_pad = jnp.zeros((128, D), jnp.float32).at[:E].set(gate_w)
    logits_pad = _router_logits(x, gate_w_pad)
    router_logits = logits_pad[:, :E]

    probs = jax.nn.softmax(router_logits, axis=-1)
    w1 = jnp.max(probs, axis=-1)
    e1 = jnp.argmax(probs, axis=-1).astype(jnp.int32)
    probs2 = jnp.where(jnp.arange(E)[None, :] == e1[:, None], -1.0, probs)
    w2 = jnp.max(probs2, axis=-1)
    e2 = jnp.argmax(probs2, axis=-1).astype(jnp.int32)
    denom = w1 + w2
    fe = jnp.stack([e1, e2], axis=1).reshape(-1)
    fw = jnp.stack([w1 / denom, w2 / denom], axis=1).reshape(-1)

    onehot = (fe[:, None] == jnp.arange(E, dtype=jnp.int32)[None, :])
    onehot = onehot.astype(jnp.int32)
    cum = jnp.cumsum(onehot, axis=0)
    rank = jnp.take_along_axis(cum - onehot, fe[:, None], axis=1)[:, 0]
    counts = cum[-1]
    padded = ((counts + BLK_M - 1) // BLK_M) * BLK_M
    pend = jnp.cumsum(padded)
    poff = pend - padded
    pos = poff[fe] + rank

    tok = (jnp.arange(N_ENTRIES, dtype=jnp.int32) // TOPK)
    row_ids = jnp.zeros((P_MAX,), jnp.int32).at[pos].set(tok)
    wrow = jnp.zeros((P_MAX,), jnp.float32).at[pos].set(fw)
    wrow_r = wrow.reshape(NUM_M, 1, BLK_M)

    total = pend[-1]
    blk_start = jnp.arange(NUM_M, dtype=jnp.int32) * BLK_M
    gid = jnp.sum((blk_start[:, None] >= pend[None, :]).astype(jnp.int32),
                  axis=1)
    gid = jnp.minimum(gid, E - 1).astype(jnp.int32)
    act = (blk_start < total).astype(jnp.int32)
    pos0, pos1 = pos[0::2], pos[1::2]

    x_s = _sc_gather_rows(x, row_ids)
    h = _grouped_gate_up(x_s, gate_proj_w, up_proj_w, wrow_r, gid, act)
    out_s = _grouped_down(h, down_proj_w, gid, act)
    final = _sc_combine(out_s, pos0, pos1)

    return (final.reshape(B, S, D), router_logits)


# speedup vs baseline: 1.0583x; 1.0583x over previous
import functools

import jax
import jax.numpy as jnp
from jax import lax
from jax.experimental import pallas as pl
from jax.experimental.pallas import tpu as pltpu
from jax.experimental.pallas import tpu_sc as plsc

B, S, D = 2, 2048, 2048
E, TOPK, DFF = 8, 2, 1408
T = B * S
N_ENTRIES = T * TOPK
BLK_M = 256
P_MAX = N_ENTRIES + E * BLK_M
NUM_M = P_MAX // BLK_M
NC, NS = 2, 16
NW = NC * NS
GATHER_CH = 16
COMB_CH = 8


def _router_logits(x, gate_w_pad):
    blk = 512

    def body(x_ref, gw_ref, out_ref):
        out_ref[...] = lax.dot_general(
            x_ref[...], gw_ref[...], (((1,), (1,)), ((), ())),
            preferred_element_type=jnp.float32)

    return pl.pallas_call(
        body,
        grid=(T // blk,),
        in_specs=[
            pl.BlockSpec((blk, D), lambda m: (m, 0)),
            pl.BlockSpec((128, D), lambda m: (0, 0)),
        ],
        out_specs=pl.BlockSpec((blk, 128), lambda m: (m, 0)),
        out_shape=jax.ShapeDtypeStruct((T, 128), jnp.float32),
    )(x, gate_w_pad)


def _sc_gather_rows(x, row_ids):
    per_w = P_MAX // NW
    nch = per_w // GATHER_CH
    mesh = plsc.VectorSubcoreMesh(core_axis_name="c", subcore_axis_name="s")

    @functools.partial(
        pl.kernel,
        out_type=jax.ShapeDtypeStruct((P_MAX, D), jnp.float32),
        mesh=mesh,
        scratch_types=[
            pltpu.VMEM((per_w,), jnp.int32),
            pltpu.VMEM((GATHER_CH, D), jnp.float32),
            pltpu.VMEM((GATHER_CH, D), jnp.float32),
            pltpu.SemaphoreType.DMA,
            pltpu.SemaphoreType.DMA,
            pltpu.SemaphoreType.DMA,
            pltpu.SemaphoreType.DMA,
        ],
    )
    def k(x_hbm, idx_hbm, out_hbm, idx_v, r0, r1, gs0, gs1, ws0, ws1):
        wid = lax.axis_index("s") * NC + lax.axis_index("c")
        base = wid * per_w
        pltpu.sync_copy(idx_hbm.at[pl.ds(base, per_w)], idx_v)

        bufs = (r0, r1)
        gsems = (gs0, gs1)
        wsems = (ws0, ws1)

        def gcopy(c):
            return pltpu.async_copy(
                x_hbm.at[idx_v.at[pl.ds(c * GATHER_CH, GATHER_CH)]],
                bufs[c % 2], gsems[c % 2])

        def wcopy(c):
            return pltpu.async_copy(
                bufs[c % 2],
                out_hbm.at[pl.ds(base + c * GATHER_CH, GATHER_CH)],
                wsems[c % 2])

        g = [None] * nch
        g[0] = gcopy(0)
        if nch > 1:
            g[1] = gcopy(1)
        for c in range(nch):
            g[c].wait()
            w = wcopy(c)
            if c + 2 < nch:
                w.wait()
                g[c + 2] = gcopy(c + 2)
            elif c + 2 >= nch:
                w.wait()

    return k(x, row_ids)


def _sc_combine(out_s, pos0, pos1):
    per_w = T // NW
    nch = per_w // COMB_CH
    mesh = plsc.VectorSubcoreMesh(core_axis_name="c", subcore_axis_name="s")

    @functools.partial(
        pl.kernel,
        out_type=jax.ShapeDtypeStruct((T, D), jnp.float32),
        mesh=mesh,
        scratch_types=[
            pltpu.VMEM((per_w,), jnp.int32),
            pltpu.VMEM((per_w,), jnp.int32),
            pltpu.VMEM((COMB_CH, D), jnp.float32),
            pltpu.VMEM((COMB_CH, D), jnp.float32),
            pltpu.VMEM((COMB_CH, D), jnp.float32),
            pltpu.VMEM((COMB_CH, D), jnp.float32),
            pltpu.SemaphoreType.DMA,
            pltpu.SemaphoreType.DMA,
            pltpu.SemaphoreType.DMA,
        ],
    )
    def k(outs_hbm, p0_hbm, p1_hbm, fin_hbm, p0_v, p1_v,
          a0, b0, a1, b1, gsem0, gsem1, wsem):
        wid = lax.axis_index("s") * NC + lax.axis_index("c")
        base = wid * per_w
        pltpu.sync_copy(p0_hbm.at[pl.ds(base, per_w)], p0_v)
        pltpu.sync_copy(p1_hbm.at[pl.ds(base, per_w)], p1_v)

        abufs = (a0, a1)
        bbufs = (b0, b1)
        gsems = (gsem0, gsem1)

        def gcopy(c):
            sl = pl.ds(c * COMB_CH, COMB_CH)
            ha = pltpu.async_copy(outs_hbm.at[p0_v.at[sl]], abufs[c % 2],
                                  gsems[c % 2])
            hb = pltpu.async_copy(outs_hbm.at[p1_v.at[sl]], bbufs[c % 2],
                                  gsems[c % 2])
            return ha, hb

        g = [None] * nch
        g[0] = gcopy(0)
        if nch > 1:
            g[1] = gcopy(1)
        for c in range(nch):
            a_v, b_v = abufs[c % 2], bbufs[c % 2]
            g[c][0].wait()
            g[c][1].wait()

            def row(r, rc):
                def col(cc, cyc):
                    sl = pl.ds(cc * 16, 16)
                    a_v[r, sl] = a_v[r, sl] + b_v[r, sl]
                    return cyc
                return lax.fori_loop(0, D // 16, col, rc)

            lax.fori_loop(0, COMB_CH, row, 0)
            w = pltpu.async_copy(
                a_v, fin_hbm.at[pl.ds(base + c * COMB_CH, COMB_CH)], wsem)
            if c + 2 < nch:
                w.wait()
                g[c + 2] = gcopy(c + 2)
            else:
                w.wait()

    return k(out_s, pos0, pos1)


def _grouped_gate_up(x_s, gate_proj_w, up_proj_w, wrow_r, gid, act):
    def body(gid_ref, act_ref, xs_ref, wg_ref, wu_ref, wr_ref, h_ref,
             wgb_ref, wub_ref, last_ref):
        m = pl.program_id(0)

        @pl.when(m == 0)
        def _():
            last_ref[0] = -1

        @pl.when(act_ref[m] == 1)
        def _():
            @pl.when(last_ref[0] != gid_ref[m])
            def _():
                wgb_ref[...] = wg_ref[0].astype(jnp.bfloat16)
                wub_ref[...] = wu_ref[0].astype(jnp.bfloat16)
                last_ref[0] = gid_ref[m]

            x = xs_ref[...].astype(jnp.bfloat16)
            g = lax.dot_general(x, wgb_ref[...], (((1,), (1,)), ((), ())),
                                preferred_element_type=jnp.float32)
            u = lax.dot_general(x, wub_ref[...], (((1,), (1,)), ((), ())),
                                preferred_element_type=jnp.float32)
            h = (g * jax.nn.sigmoid(g)) * u
            h_ref[...] = (h * wr_ref[0, 0, :][:, None]).astype(jnp.bfloat16)

    grid_spec = pltpu.PrefetchScalarGridSpec(
        num_scalar_prefetch=2,
        grid=(NUM_M,),
        in_specs=[
            pl.BlockSpec(
                (BLK_M, D),
                lambda m, gid, act: (jnp.where(act[m] == 1, m, 0), 0)),
            pl.BlockSpec((1, DFF, D), lambda m, gid, act: (gid[m], 0, 0)),
            pl.BlockSpec((1, DFF, D), lambda m, gid, act: (gid[m], 0, 0)),
            pl.BlockSpec((1, 1, BLK_M), lambda m, gid, act: (m, 0, 0)),
        ],
        out_specs=pl.BlockSpec((BLK_M, DFF), lambda m, gid, act: (m, 0)),
        scratch_shapes=[
            pltpu.VMEM((DFF, D), jnp.bfloat16),
            pltpu.VMEM((DFF, D), jnp.bfloat16),
            pltpu.SMEM((1,), jnp.int32),
        ],
    )
    return pl.pallas_call(
        body,
        grid_spec=grid_spec,
        out_shape=jax.ShapeDtypeStruct((P_MAX, DFF), jnp.bfloat16),
        compiler_params=pltpu.CompilerParams(vmem_limit_bytes=100 * 1024 * 1024),
    )(gid, act, x_s, gate_proj_w, up_proj_w, wrow_r)


def _grouped_down(h, down_proj_w, gid, act):
    def body(gid_ref, act_ref, h_ref, wd_ref, out_ref, wdb_ref, last_ref):
        m = pl.program_id(0)

        @pl.when(m == 0)
        def _():
            last_ref[0] = -1

        @pl.when(act_ref[m] == 1)
        def _():
            @pl.when(last_ref[0] != gid_ref[m])
            def _():
                wdb_ref[...] = wd_ref[0].astype(jnp.bfloat16)
                last_ref[0] = gid_ref[m]

            out_ref[...] = lax.dot_general(
                h_ref[...], wdb_ref[...], (((1,), (1,)), ((), ())),
                preferred_element_type=jnp.float32)

    grid_spec = pltpu.PrefetchScalarGridSpec(
        num_scalar_prefetch=2,
        grid=(NUM_M,),
        in_specs=[
            pl.BlockSpec((BLK_M, DFF),
                         lambda m, gid, act: (jnp.where(act[m] == 1, m, 0), 0)),
            pl.BlockSpec((1, D, DFF), lambda m, gid, act: (gid[m], 0, 0)),
        ],
        out_specs=pl.BlockSpec((BLK_M, D), lambda m, gid, act: (m, 0)),
        scratch_shapes=[
            pltpu.VMEM((D, DFF), jnp.bfloat16),
            pltpu.SMEM((1,), jnp.int32),
        ],
    )
    return pl.pallas_call(
        body,
        grid_spec=grid_spec,
        out_shape=jax.ShapeDtypeStruct((P_MAX, D), jnp.float32),
    )(gid, act, h, down_proj_w)


def kernel(hidden_states, gate_w, gate_proj_w, up_proj_w, down_proj_w):
    x = hidden_states.reshape(-1, D)

    gate_w_pad = jnp.zeros((128, D), jnp.float32).at[:E].set(gate_w)
    logits_pad = _router_logits(x, gate_w_pad)
    router_logits = logits_pad[:, :E]

    probs = jax.nn.softmax(router_logits, axis=-1)
    w1 = jnp.max(probs, axis=-1)
    e1 = jnp.argmax(probs, axis=-1).astype(jnp.int32)
    probs2 = jnp.where(jnp.arange(E)[None, :] == e1[:, None], -1.0, probs)
    w2 = jnp.max(probs2, axis=-1)
    e2 = jnp.argmax(probs2, axis=-1).astype(jnp.int32)
    denom = w1 + w2
    fe = jnp.stack([e1, e2], axis=1).reshape(-1)
    fw = jnp.stack([w1 / denom, w2 / denom], axis=1).reshape(-1)

    onehot = (fe[:, None] == jnp.arange(E, dtype=jnp.int32)[None, :])
    onehot = onehot.astype(jnp.int32)
    cum = jnp.cumsum(onehot, axis=0)
    rank = jnp.sum(jnp.where(onehot == 1, cum - 1, 0), axis=1)
    counts = cum[-1]
    padded = ((counts + BLK_M - 1) // BLK_M) * BLK_M
    pend = jnp.cumsum(padded)
    poff = pend - padded
    pos = poff[fe] + rank

    tok = (jnp.arange(N_ENTRIES, dtype=jnp.int32) // TOPK)
    row_ids = jnp.zeros((P_MAX,), jnp.int32).at[pos].set(
        tok, unique_indices=True, mode="promise_in_bounds")
    wrow = jnp.zeros((P_MAX,), jnp.float32).at[pos].set(
        fw, unique_indices=True, mode="promise_in_bounds")
    wrow_r = wrow.reshape(NUM_M, 1, BLK_M)

    total = pend[-1]
    blk_start = jnp.arange(NUM_M, dtype=jnp.int32) * BLK_M
    gid = jnp.sum((blk_start[:, None] >= pend[None, :]).astype(jnp.int32),
                  axis=1)
    gid = jnp.minimum(gid, E - 1).astype(jnp.int32)
    act = (blk_start < total).astype(jnp.int32)
    pos0, pos1 = pos[0::2], pos[1::2]

    x_s = _sc_gather_rows(x, row_ids)
    h = _grouped_gate_up(x_s, gate_proj_w, up_proj_w, wrow_r, gid, act)
    out_s = _grouped_down(h, down_proj_w, gid, act)
    final = _sc_combine(out_s, pos0, pos1)

    return (final.reshape(B, S, D), router_logits)
